# baseline (speedup 1.0000x reference)
import functools

import jax
import jax.numpy as jnp
from jax import lax
from jax.experimental import pallas as pl
from jax.experimental.pallas import tpu as pltpu
from jax.experimental.pallas import tpu_sc as plsc

BATCH = 16384
NC = 2
NS = 16
NW = NC * NS
B_PER_W = BATCH // NW
B_HALF = B_PER_W // 2

N_ROWS = 1000000
WIN = 1024
NRING = 3
NFULL = 976
WPH = NFULL // 2
TAIL0 = NFULL * WIN
TAIL_DMA = 512
LAST0 = TAIL0 + TAIL_DMA

D_USER, D_GENDER, D_GENRE, D_ARTIST = 64, 16, 32, 64

GCAP = 2048
GFLUSH = 32
CHG = 256
NCH = GCAP // CHG

_sc_mesh = plsc.VectorSubcoreMesh(core_axis_name="c", subcore_axis_name="s")

_I16 = lambda: lax.iota(jnp.int32, 16)
_NEG = -2147483647 - 1


def _bcast(x):
    return jnp.full((16,), x, jnp.int32)


def _sort_by_window(idxbuf, sorted_pk, cursor, starts):
    zeros = jnp.zeros((16,), jnp.int32)

    def zero_body(i, carry):
        cursor[pl.ds(i * 16, 16)] = zeros
        return carry

    lax.fori_loop(0, 64, zero_body, 0)

    def hist_body(b, carry):
        v = idxbuf[pl.ds(b * 16, 16)]
        bucket = lax.shift_right_logical(v, 10)
        cnt, last = plsc.scan_count(bucket)
        base = plsc.load_gather(cursor, [bucket])
        plsc.store_scatter(cursor, [bucket], base + cnt, mask=last)
        return carry

    lax.fori_loop(0, BATCH // 16, hist_body, 0)

    starts[0] = 0

    def pfx_body(i, running):
        v = cursor[pl.ds(i * 16, 16)]
        c = plsc.cumsum(v)
        runv = _bcast(running)
        cursor[pl.ds(i * 16, 16)] = runv + c - v
        inc = runv + c
        for l in range(16):
            starts[i * 16 + l + 1] = inc[l]
        return running + c[15]

    lax.fori_loop(0, 63, pfx_body, 0)

    def scat_body(b, carry):
        v = idxbuf[pl.ds(b * 16, 16)]
        bucket = lax.shift_right_logical(v, 10)
        cnt, last = plsc.scan_count(bucket)
        base = plsc.load_gather(cursor, [bucket])
        slot = base + cnt - 1
        pkv = lax.shift_left(_bcast(b * 16) + _I16(), 10) | (v & 1023)
        plsc.store_scatter(sorted_pk, [slot], pkv)
        plsc.store_scatter(cursor, [bucket], base + cnt, mask=last)
        return carry

    lax.fori_loop(0, BATCH // 16, scat_body, 0)


def _serve(tbl_hbm, pos_list, val_list, wid, tr, h,
           sorted_pk, bufs, dsems, tailtab, posbuf, valbuf, starts, ssem):
    i16 = _I16()
    tr8 = tr * 8
    nbuf = len(bufs)

    def flush(ngrp):
        gbase = (ngrp - GFLUSH) * 16
        vbase = (ngrp - GFLUSH) * 128
        c1 = pltpu.async_copy(
            posbuf, pos_list.at[pl.ds(wid * GCAP * 16 + gbase, 16 * GFLUSH)],
            ssem)
        c2 = pltpu.async_copy(
            valbuf, val_list.at[pl.ds(wid * GCAP * 128 + vbase, 128 * GFLUSH)],
            ssem)
        c1.wait()
        c2.wait()
        return jnp.int32(0)

    def hits_for(w, carry, buf):
        s = starts[w]
        e = starts[w + 1]

        def grp(g, carry2):
            ngrp, nb = carry2
            pk = sorted_pk[pl.ds(pl.multiple_of(g * 16, 16), 16)]
            slots = _bcast(g * 16) + i16
            inm = (slots >= s) & (slots < e)
            pos = lax.shift_right_logical(pk, 10)
            lrel = pk & 1023
            posv = jnp.where(inm, pos, jnp.int32(-1))
            posbuf[pl.ds(nb * 16, 16)] = posv
            for s8 in range(8):
                vals = plsc.load_gather(buf, [_bcast(s8), lrel])
                valbuf[pl.ds(nb * 128 + s8 * 16, 16)] = vals
            ngrp = ngrp + 1
            nb = nb + 1
            nb = lax.cond(nb == GFLUSH, lambda n: flush(ngrp),
                          lambda n: n, nb)
            return ngrp, nb

        return lax.fori_loop(s // 16, (e + 15) // 16, grp, carry)

    def fire(w, buf, sem):
        lane0 = pl.multiple_of(w * WIN, WIN)
        return pltpu.async_copy(
            tbl_hbm.at[pl.ds(tr8, 8), pl.ds(lane0, WIN)], buf, sem)

    def wait_buf(buf, sem):
        pltpu.make_async_copy(
            tbl_hbm.at[pl.ds(0, 8), pl.ds(0, WIN)], buf, sem).wait()

    wlo = h * WPH
    for k in range(nbuf):
        fire(wlo + k, bufs[k], dsems[k])

    def ring_body(p, carry):
        w0 = wlo + p * nbuf
        for k in range(nbuf):
            wait_buf(bufs[k], dsems[k])
            carry = hits_for(w0 + k, carry, bufs[k])
            wn = w0 + k + nbuf
            wn = lax.select(wn < NFULL, wn, 0)
            fire(wn, bufs[k], dsems[k])
        return carry

    nring = WPH // nbuf
    rem = WPH - nring * nbuf
    carry = lax.fori_loop(0, nring, ring_body,
                          (jnp.int32(0), jnp.int32(0)))
    for k in range(nbuf):
        wait_buf(bufs[k], dsems[k])
        if k < rem:
            carry = hits_for(wlo + nring * nbuf + k, carry, bufs[k])

    def tail_body(_, carry2):
        pltpu.sync_copy(
            tbl_hbm.at[pl.ds(tr8, 8), pl.ds(TAIL0, TAIL_DMA)],
            bufs[0].at[:, pl.ds(0, TAIL_DMA)])

        def tail_cp(s8, c3):
            for cb in range(4):
                cols = _I16() + cb * 16
                v = plsc.load_gather(tailtab, [_bcast(tr8 + s8), cols])
                plsc.store_scatter(bufs[0], [_bcast(s8), cols + TAIL_DMA], v)
            return c3

        lax.fori_loop(0, 8, tail_cp, 0)
        return hits_for(NFULL, carry2, bufs[0])

    ngrp, nb = lax.fori_loop(0, lax.select(h == 1, 1, 0), tail_body, carry)

    def pad_body(i, c2):
        posbuf[pl.ds((nb + i) * 16, 16)] = jnp.full((16,), -1, jnp.int32)
        return c2

    lax.fori_loop(0, GFLUSH - nb, pad_body, 0)
    flush(ngrp + (GFLUSH - nb))
    return jnp.int32(0)


@functools.partial(
    pl.kernel,
    out_type=(
        jax.ShapeDtypeStruct((NW * GCAP * 16,), jnp.int32),
        jax.ShapeDtypeStruct((NW * GCAP * 128,), jnp.float32),
        jax.ShapeDtypeStruct((NW * 8,), jnp.int32),
        jax.ShapeDtypeStruct((D_GENDER, BATCH), jnp.float32),
        jax.ShapeDtypeStruct((D_GENRE, BATCH), jnp.float32),
    ),
    mesh=_sc_mesh,
    scratch_types=(
        pltpu.VMEM((BATCH,), jnp.int32),
        pltpu.VMEM((BATCH,), jnp.int32),
        pltpu.VMEM((8, WIN), jnp.float32),
        pltpu.VMEM((8, WIN), jnp.float32),
        pltpu.VMEM((8, WIN), jnp.float32),
        pltpu.VMEM((64, 128), jnp.float32),
        pltpu.VMEM((GFLUSH * 16,), jnp.int32),
        pltpu.VMEM((GFLUSH * 128,), jnp.float32),
        pltpu.VMEM((1024,), jnp.int32),
        pltpu.SMEM((1024,), jnp.int32),
        pltpu.VMEM((D_GENRE, 1024), jnp.float32),
        pltpu.VMEM((D_GENDER, 128), jnp.float32),
        pltpu.VMEM((D_GENDER, B_HALF), jnp.float32),
        pltpu.VMEM((D_GENRE, B_HALF), jnp.float32),
        pltpu.VMEM((B_PER_W,), jnp.int32),
        pltpu.VMEM((B_PER_W,), jnp.int32),
        pltpu.SemaphoreType.DMA,
        pltpu.SemaphoreType.DMA,
        pltpu.SemaphoreType.DMA,
        pltpu.SemaphoreType.DMA,
    ),
    compiler_params=pltpu.CompilerParams(needs_layout_passes=False),
)
def _sc_harvest(ids_hbm, gid_hbm, gnr_hbm, ut_hbm, at_hbm,
                gt_hbm, gnt_hbm, tails_hbm,
                pos_list, val_list, gcnt, out_g, out_gn,
                idxbuf, sorted_pk, buf0, buf1, buf2, tailtab,
                posbuf, valbuf, cursor, starts,
                gntab, gtab, gstage, gnstage, gidx, gnidx,
                d0, d1, d2, ssem):
    s_id = lax.axis_index("s")
    c_id = lax.axis_index("c")
    t = lax.bitwise_and(s_id, 1)
    tr = lax.shift_right_logical(s_id, 1)
    h = c_id
    wid = s_id * NC + c_id

    own = pl.ds(wid * B_PER_W, B_PER_W)
    pltpu.sync_copy(gid_hbm.at[own], gidx)
    pltpu.sync_copy(gnr_hbm.at[own], gnidx)
    pltpu.sync_copy(gnt_hbm, gntab)
    pltpu.sync_copy(gt_hbm, gtab)

    for half in range(2):
        hb = half * B_HALF

        def sel_body(b, carry):
            gv = gidx[pl.ds(hb + b * 16, 16)]
            gnv = gnidx[pl.ds(hb + b * 16, 16)]
            i16 = _I16()
            for l in range(16):
                ib = _bcast(b * 16 + l)
                coln = _bcast(gnv[l])
                v_lo = plsc.load_gather(gntab, [i16, coln])
                v_hi = plsc.load_gather(gntab, [i16 + 16, coln])
                plsc.store_scatter(gnstage, [i16, ib], v_lo)
                plsc.store_scatter(gnstage, [i16 + 16, ib], v_hi)
                colg = _bcast(gv[l])
                vg = plsc.load_gather(gtab, [i16, colg])
                plsc.store_scatter(gstage, [i16, ib], vg)
            return carry

        lax.fori_loop(0, B_HALF // 16, sel_body, 0)
        out_cols = pl.ds(
            pl.multiple_of(wid * B_PER_W + hb, B_HALF), B_HALF)
        pltpu.sync_copy(gnstage, out_gn.at[:, out_cols])
        pltpu.sync_copy(gstage, out_g.at[:, out_cols])

    pltpu.sync_copy(ids_hbm.at[pl.ds(t * BATCH, BATCH)], idxbuf)
    pltpu.sync_copy(tails_hbm.at[pl.ds(t * 64, 64)], tailtab)
    _sort_by_window(idxbuf, sorted_pk, cursor, starts)

    bufs = (buf0, buf1, buf2)
    dsems = (d0, d1, d2)

    def serve_u(_, carry):
        return _serve(ut_hbm, pos_list, val_list, wid, tr, h,
                      sorted_pk, bufs, dsems, tailtab, posbuf, valbuf,
                      starts, ssem)

    def serve_a(_, carry):
        return _serve(at_hbm, pos_list, val_list, wid, tr, h,
                      sorted_pk, bufs, dsems, tailtab, posbuf, valbuf,
                      starts, ssem)

    lax.fori_loop(0, 1 - t, serve_u, jnp.int32(0))
    lax.fori_loop(0, t, serve_a, jnp.int32(0))

    def cnt_body(w, acc):
        s = starts[w]
        e = starts[w + 1]
        return acc + (e + 15) // 16 - s // 16

    wlo = h * WPH
    gtot = lax.fori_loop(wlo, wlo + WPH, cnt_body, jnp.int32(0))
    gtot = lax.fori_loop(
        0, lax.select(h == 1, 1, 0),
        lambda _, acc: cnt_body(NFULL, acc), gtot)
    gtot = (gtot + GFLUSH - 1) // GFLUSH * GFLUSH
    cursor[pl.ds(0, 16)] = _bcast(gtot)
    pltpu.sync_copy(cursor.at[pl.ds(0, 8)], gcnt.at[pl.ds(wid * 8, 8)])


NSLOT = 16


@functools.partial(
    pl.kernel,
    out_type=(
        jax.ShapeDtypeStruct((BATCH * D_USER,), jnp.float32),
        jax.ShapeDtypeStruct((BATCH * D_ARTIST,), jnp.float32),
    ),
    mesh=_sc_mesh,
    scratch_types=(
        pltpu.VMEM((B_PER_W * 64,), jnp.float32),
        pltpu.VMEM((CHG * 16,), jnp.int32),
        pltpu.VMEM((CHG * 16,), jnp.int32),
        pltpu.VMEM((NSLOT * 128,), jnp.float32),
        pltpu.VMEM((NSLOT * 16,), jnp.int32),
        pltpu.VMEM((NW * 8,), jnp.int32),
        pltpu.SemaphoreType.DMA,
        pltpu.SemaphoreType.DMA,
        pltpu.SemaphoreType.DMA,
    ),
    compiler_params=pltpu.CompilerParams(needs_layout_passes=False),
)
def _sc_distribute(pos_list, val_list, gcnt_hbm,
                   out_u, out_a,
                   rows, pchA, pchB, ring, stash, cnts, semA, semB, rsem):
    s_id = lax.axis_index("s")
    c_id = lax.axis_index("c")
    wid = s_id * NC + c_id
    base = wid * B_PER_W
    i16 = _I16()

    pltpu.sync_copy(gcnt_hbm, cnts)

    def fire_pos(src, cb, pch, sem):
        cbc = lax.select(cb < NCH, cb, NCH - 1)
        pltpu.async_copy(
            pos_list.at[pl.ds(src * (GCAP * 16) + cbc * (CHG * 16),
                              CHG * 16)], pch, sem)

    def wait_pos(pch, sem):
        pltpu.make_async_copy(
            pos_list.at[pl.ds(0, CHG * 16)], pch, sem).wait()

    def proc_oldest(c):
        nfired, nproc = c
        pltpu.make_async_copy(
            val_list.at[pl.ds(0, 128)], ring.at[pl.ds(0, 128)], rsem).wait()
        slot = lax.bitwise_and(nproc, NSLOT - 1)
        ov = stash[pl.ds(slot * 16, 16)]
        m = ov >= 0

        def w_cond(m2):
            return plsc.all_reduce_population_count(m2)[0] > 0

        def w_body(m2):
            ffs = plsc.all_reduce_ffs(m2)
            l0 = ffs[0]
            offsc = jnp.max(jnp.where(i16 == ffs, ov, jnp.int32(_NEG)))
            v8 = plsc.load_gather(
                ring, [_bcast(slot * 128) + _bcast(l0) + i16 * 16],
                mask=i16 < 8)
            plsc.store_scatter(rows, [_bcast(offsc) + i16], v8,
                               mask=i16 < 8)
            return m2 & (i16 != ffs)

        lax.while_loop(w_cond, w_body, m)
        return nfired, nproc + 1

    def drain(c):
        def d_cond(cc):
            nf, np_ = cc
            return np_ < nf

        return lax.while_loop(d_cond, proc_oldest, c)

    def do_table(t, out_flat):
        def src_body(k, carry):
            src_s = t + 2 * lax.shift_right_logical(k, 1)
            src = src_s * NC + lax.bitwise_and(k, 1)
            trf = lax.shift_right_logical(src_s, 1) * 8
            ng = plsc.load_gather(cnts, [_bcast(src * 8)])[0]
            nch = (ng + CHG - 1) // CHG
            vbase = src * (GCAP * 128)

            def process(cb, pch, c2):
                glim = lax.max(
                    jnp.int32(0), lax.min(jnp.int32(CHG), ng - cb * CHG))

                def g_body(g, c3):
                    pv = pch[pl.ds(pl.multiple_of(g * 16, 16), 16)]
                    m = lax.shift_right_arithmetic(pv, 9) == wid

                    def fire_grp(_, c4):
                        c4 = lax.cond(
                            c4[0] - c4[1] == NSLOT, proc_oldest,
                            lambda cc: cc, c4)
                        nfired, nproc = c4
                        slot = lax.bitwise_and(nfired, NSLOT - 1)
                        offv = jnp.where(m, (pv - base) * 64 + trf,
                                         jnp.int32(-1))
                        stash[pl.ds(slot * 16, 16)] = offv
                        pltpu.async_copy(
                            val_list.at[pl.ds(
                                vbase + (cb * CHG + g) * 128, 128)],
                            ring.at[pl.ds(slot * 128, 128)], rsem)
                        return nfired + 1, nproc

                    anyhit = plsc.all_reduce_population_count(m)[0] > 0
                    return lax.fori_loop(
                        0, lax.select(anyhit, 1, 0), fire_grp, c3)

                return lax.fori_loop(0, glim, g_body, c2)

            fire_pos(src, jnp.int32(0), pchA, semA)

            def pair_body(p, c2):
                c0 = p * 2
                fire_pos(src, c0 + 1, pchB, semB)
                wait_pos(pchA, semA)
                c2 = process(c0, pchA, c2)
                fire_pos(src, c0 + 2, pchA, semA)
                wait_pos(pchB, semB)
                c2 = process(c0 + 1, pchB, c2)
                return c2

            carry2 = lax.fori_loop(0, (nch + 1) // 2, pair_body, carry)
            wait_pos(pchA, semA)
            return carry2

        carry = lax.fori_loop(0, 16, src_body, (jnp.int32(0), jnp.int32(0)))
        carry = drain(carry)
        pltpu.sync_copy(rows, out_flat.at[pl.ds(base * 64, B_PER_W * 64)])
        return carry

    c = do_table(0, out_u)
    del c
    do_table(1, out_a)


def _mlp_body(x_ref, w1_ref, b1_ref, w2_ref, b2_ref, o_ref):
    hh = lax.dot_general(x_ref[:], w1_ref[:], (((1,), (1,)), ((), ())),
                         preferred_element_type=jnp.float32)
    hh = jnp.maximum(hh + b1_ref[:], 0.0)
    o = lax.dot_general(hh, w2_ref[:], (((1,), (1,)), ((), ())),
                        preferred_element_type=jnp.float32)
    o_ref[:] = o + b2_ref[:]


_MLP_BLK = 1024


@jax.jit
def _mlp(audio_features, W1, b1, W2, b2):
    grid = (BATCH // _MLP_BLK,)
    return pl.pallas_call(
        _mlp_body,
        grid=grid,
        in_specs=[
            pl.BlockSpec((_MLP_BLK, 128), lambda i: (i, 0)),
            pl.BlockSpec((256, 128), lambda i: (0, 0)),
            pl.BlockSpec((1, 256), lambda i: (0, 0)),
            pl.BlockSpec((128, 256), lambda i: (0, 0)),
            pl.BlockSpec((1, 128), lambda i: (0, 0)),
        ],
        out_specs=pl.BlockSpec((_MLP_BLK, 128), lambda i: (i, 0)),
        out_shape=jax.ShapeDtypeStruct((BATCH, 128), jnp.float32),
    )(audio_features, W1, b1.reshape(1, 256), W2, b2.reshape(1, 128))


@jax.jit
def kernel(user_ids, genders, genres, artist_ids, audio_features,
           user_table, gender_table, genre_table, artist_table,
           W1, b1, W2, b2):
    ut_t = user_table.T
    at_t = artist_table.T
    ids_ua = jnp.concatenate(
        [user_ids.astype(jnp.int32), artist_ids.astype(jnp.int32)])
    tails = jnp.concatenate(
        [jnp.pad(ut_t[:, LAST0:], ((0, 0), (0, 64))),
         jnp.pad(at_t[:, LAST0:], ((0, 0), (0, 64)))], axis=0)
    gnt_pad = jnp.pad(genre_table.T, ((0, 0), (0, 24)))
    gt_pad = jnp.pad(gender_table.T, ((0, 0), (0, 124)))

    pos_list, val_list, gcnt, g_t, gn_t = _sc_harvest(
        ids_ua, genders.astype(jnp.int32), genres.astype(jnp.int32),
        ut_t, at_t, gt_pad, gnt_pad, tails)

    u_flat, a_flat = _sc_distribute(pos_list, val_list, gcnt)

    audio_emb = _mlp(audio_features, W1, b1, W2, b2)
    return (u_flat.reshape(BATCH, D_USER),
            g_t.T,
            gn_t.T,
            a_flat.reshape(BATCH, D_ARTIST),
            audio_emb)


# speedup vs baseline: 2.0685x; 2.0685x over previous
import functools

import jax
import jax.numpy as jnp
from jax import lax
from jax.experimental import pallas as pl
from jax.experimental.pallas import tpu as pltpu
from jax.experimental.pallas import tpu_sc as plsc

BATCH = 16384
NC = 2
NS = 16
NW = NC * NS
B_PER_W = BATCH // NW
B_HALF = B_PER_W // 2

N_ROWS = 1000000
WIN = 1024
NRING = 3
NFULL = 976
WPH = NFULL // 2
TAIL0 = NFULL * WIN
TAIL_DMA = 512
LAST0 = TAIL0 + TAIL_DMA

D_USER, D_GENDER, D_GENRE, D_ARTIST = 64, 16, 32, 64

GCAP = 2048
GFLUSH = 32
DCAP = BATCH + 512
DCHW = 2048

_sc_mesh = plsc.VectorSubcoreMesh(core_axis_name="c", subcore_axis_name="s")

_I16 = lambda: lax.iota(jnp.int32, 16)
_NEG = -2147483647 - 1


def _bcast(x):
    return jnp.full((16,), x, jnp.int32)


def _sort_by_window(idxbuf, sorted_pk, cursor, starts):
    zeros = jnp.zeros((16,), jnp.int32)

    def zero_body(i, carry):
        cursor[pl.ds(i * 16, 16)] = zeros
        return carry

    lax.fori_loop(0, 64, zero_body, 0)

    def hist_body(b, carry):
        v = idxbuf[pl.ds(b * 16, 16)]
        bucket = lax.shift_right_logical(v, 10)
        cnt, last = plsc.scan_count(bucket)
        base = plsc.load_gather(cursor, [bucket])
        plsc.store_scatter(cursor, [bucket], base + cnt, mask=last)
        return carry

    lax.fori_loop(0, BATCH // 16, hist_body, 0)

    starts[0] = 0

    def pfx_body(i, running):
        v = cursor[pl.ds(i * 16, 16)]
        c = plsc.cumsum(v)
        runv = _bcast(running)
        cursor[pl.ds(i * 16, 16)] = runv + c - v
        inc = runv + c
        for l in range(16):
            starts[i * 16 + l + 1] = inc[l]
        return running + c[15]

    lax.fori_loop(0, 63, pfx_body, 0)

    def scat_body(b, carry):
        v = idxbuf[pl.ds(b * 16, 16)]
        bucket = lax.shift_right_logical(v, 10)
        cnt, last = plsc.scan_count(bucket)
        base = plsc.load_gather(cursor, [bucket])
        slot = base + cnt - 1
        pkv = lax.shift_left(_bcast(b * 16) + _I16(), 10) | (v & 1023)
        plsc.store_scatter(sorted_pk, [slot], pkv)
        plsc.store_scatter(cursor, [bucket], base + cnt, mask=last)
        return carry

    lax.fori_loop(0, BATCH // 16, scat_body, 0)


def _serve(tbl_hbm, val_list, wid, tr, h,
           sorted_pk, bufs, dsems, tailtab, valbuf, starts, ssem):
    i16 = _I16()
    tr8 = tr * 8
    nbuf = len(bufs)

    def flush(ngrp):
        vbase = (ngrp - GFLUSH) * 128
        pltpu.async_copy(
            valbuf, val_list.at[pl.ds(wid * GCAP * 128 + vbase, 128 * GFLUSH)],
            ssem).wait()
        return jnp.int32(0)

    def hits_for(w, carry, buf):
        s = starts[w]
        e = starts[w + 1]

        def grp(g, carry2):
            ngrp, nb = carry2
            pk = sorted_pk[pl.ds(pl.multiple_of(g * 16, 16), 16)]
            lrel = pk & 1023
            for s8 in range(8):
                vals = plsc.load_gather(buf, [_bcast(s8), lrel])
                valbuf[pl.ds(nb * 128 + s8 * 16, 16)] = vals
            ngrp = ngrp + 1
            nb = nb + 1
            nb = lax.cond(nb == GFLUSH, lambda n: flush(ngrp),
                          lambda n: n, nb)
            return ngrp, nb

        return lax.fori_loop(s // 16, (e + 15) // 16, grp, carry)

    def fire(w, buf, sem):
        lane0 = pl.multiple_of(w * WIN, WIN)
        return pltpu.async_copy(
            tbl_hbm.at[pl.ds(tr8, 8), pl.ds(lane0, WIN)], buf, sem)

    def wait_buf(buf, sem):
        pltpu.make_async_copy(
            tbl_hbm.at[pl.ds(0, 8), pl.ds(0, WIN)], buf, sem).wait()

    wlo = h * WPH
    for k in range(nbuf):
        fire(wlo + k, bufs[k], dsems[k])

    def ring_body(p, carry):
        w0 = wlo + p * nbuf
        for k in range(nbuf):
            wait_buf(bufs[k], dsems[k])
            carry = hits_for(w0 + k, carry, bufs[k])
            wn = w0 + k + nbuf
            wn = lax.select(wn < NFULL, wn, 0)
            fire(wn, bufs[k], dsems[k])
        return carry

    nring = WPH // nbuf
    rem = WPH - nring * nbuf
    carry = lax.fori_loop(0, nring, ring_body,
                          (jnp.int32(0), jnp.int32(0)))
    for k in range(nbuf):
        wait_buf(bufs[k], dsems[k])
        if k < rem:
            carry = hits_for(wlo + nring * nbuf + k, carry, bufs[k])

    def tail_body(_, carry2):
        pltpu.sync_copy(
            tbl_hbm.at[pl.ds(tr8, 8), pl.ds(TAIL0, TAIL_DMA)],
            bufs[0].at[:, pl.ds(0, TAIL_DMA)])

        def tail_cp(s8, c3):
            for cb in range(4):
                cols = _I16() + cb * 16
                v = plsc.load_gather(tailtab, [_bcast(tr8 + s8), cols])
                plsc.store_scatter(bufs[0], [_bcast(s8), cols + TAIL_DMA], v)
            return c3

        lax.fori_loop(0, 8, tail_cp, 0)
        return hits_for(NFULL, carry2, bufs[0])

    ngrp, nb = lax.fori_loop(0, lax.select(h == 1, 1, 0), tail_body, carry)

    flush(ngrp + (GFLUSH - nb))
    return jnp.int32(0)


@functools.partial(
    pl.kernel,
    out_type=(
        jax.ShapeDtypeStruct((NW * GCAP * 128,), jnp.float32),
        jax.ShapeDtypeStruct((NW * DCAP + DCHW,), jnp.int32),
        jax.ShapeDtypeStruct((NW * 64,), jnp.int32),
        jax.ShapeDtypeStruct((D_GENDER, BATCH), jnp.float32),
        jax.ShapeDtypeStruct((D_GENRE, BATCH), jnp.float32),
    ),
    mesh=_sc_mesh,
    scratch_types=(
        pltpu.VMEM((DCAP,), jnp.int32),
        pltpu.VMEM((BATCH,), jnp.int32),
        pltpu.VMEM((8, WIN), jnp.float32),
        pltpu.VMEM((8, WIN), jnp.float32),
        pltpu.VMEM((8, WIN), jnp.float32),
        pltpu.VMEM((64, 128), jnp.float32),
        pltpu.VMEM((GFLUSH * 16,), jnp.int32),
        pltpu.VMEM((GFLUSH * 128,), jnp.float32),
        pltpu.VMEM((1024,), jnp.int32),
        pltpu.SMEM((1024,), jnp.int32),
        pltpu.VMEM((D_GENRE, 1024), jnp.float32),
        pltpu.VMEM((D_GENDER, 128), jnp.float32),
        pltpu.VMEM((D_GENDER, B_HALF), jnp.float32),
        pltpu.VMEM((D_GENRE, B_HALF), jnp.float32),
        pltpu.VMEM((B_PER_W,), jnp.int32),
        pltpu.VMEM((B_PER_W,), jnp.int32),
        pltpu.SemaphoreType.DMA,
        pltpu.SemaphoreType.DMA,
        pltpu.SemaphoreType.DMA,
        pltpu.SemaphoreType.DMA,
    ),
    compiler_params=pltpu.CompilerParams(needs_layout_passes=False),
)
def _sc_harvest(ids_hbm, gid_hbm, gnr_hbm, ut_hbm, at_hbm,
                gt_hbm, gnt_hbm, tails_hbm,
                val_list, dperm_list, dmeta, out_g, out_gn,
                idxbuf, sorted_pk, buf0, buf1, buf2, tailtab,
                posbuf, valbuf, cursor, starts,
                gntab, gtab, gstage, gnstage, gidx, gnidx,
                d0, d1, d2, ssem):
    s_id = lax.axis_index("s")
    c_id = lax.axis_index("c")
    t = lax.bitwise_and(s_id, 1)
    tr = lax.shift_right_logical(s_id, 1)
    h = c_id
    wid = s_id * NC + c_id

    own = pl.ds(wid * B_PER_W, B_PER_W)
    pltpu.sync_copy(gid_hbm.at[own], gidx)
    pltpu.sync_copy(gnr_hbm.at[own], gnidx)
    pltpu.sync_copy(gnt_hbm, gntab)
    pltpu.sync_copy(gt_hbm, gtab)

    for half in range(2):
        hb = half * B_HALF

        def sel_body(b, carry):
            gv = gidx[pl.ds(hb + b * 16, 16)]
            gnv = gnidx[pl.ds(hb + b * 16, 16)]
            i16 = _I16()
            for l in range(16):
                ib = _bcast(b * 16 + l)
                coln = _bcast(gnv[l])
                v_lo = plsc.load_gather(gntab, [i16, coln])
                v_hi = plsc.load_gather(gntab, [i16 + 16, coln])
                plsc.store_scatter(gnstage, [i16, ib], v_lo)
                plsc.store_scatter(gnstage, [i16 + 16, ib], v_hi)
                colg = _bcast(gv[l])
                vg = plsc.load_gather(gtab, [i16, colg])
                plsc.store_scatter(gstage, [i16, ib], vg)
            return carry

        lax.fori_loop(0, B_HALF // 16, sel_body, 0)
        out_cols = pl.ds(
            pl.multiple_of(wid * B_PER_W + hb, B_HALF), B_HALF)
        pltpu.sync_copy(gnstage, out_gn.at[:, out_cols])
        pltpu.sync_copy(gstage, out_g.at[:, out_cols])

    pltpu.sync_copy(ids_hbm.at[pl.ds(t * BATCH, BATCH)], idxbuf.at[pl.ds(0, BATCH)])
    pltpu.sync_copy(tails_hbm.at[pl.ds(t * 64, 64)], tailtab)
    _sort_by_window(idxbuf, sorted_pk, cursor, starts)

    bufs = (buf0, buf1, buf2)
    dsems = (d0, d1, d2)

    def serve_u(_, carry):
        return _serve(ut_hbm, val_list, wid, tr, h,
                      sorted_pk, bufs, dsems, tailtab, valbuf,
                      starts, ssem)

    def serve_a(_, carry):
        return _serve(at_hbm, val_list, wid, tr, h,
                      sorted_pk, bufs, dsems, tailtab, valbuf,
                      starts, ssem)

    lax.fori_loop(0, 1 - t, serve_u, jnp.int32(0))
    lax.fori_loop(0, t, serve_a, jnp.int32(0))

    i16b = _I16()
    wlo = h * WPH
    slo = starts[wlo]
    shi = lax.select(h == 1, jnp.int32(BATCH), starts[wlo + WPH])

    zeros2 = jnp.zeros((16,), jnp.int32)
    cursor[pl.ds(0, 16)] = zeros2
    cursor[pl.ds(16, 16)] = zeros2

    def dh_body(g, carry):
        sl = _bcast(g * 16) + i16b
        inm = (sl >= slo) & (sl < shi)
        v = sorted_pk[pl.ds(pl.multiple_of(g * 16, 16), 16)]
        d = lax.shift_right_logical(v, 19)
        cnt, last = plsc.scan_count(d, inm)
        base = plsc.load_gather(cursor, [d])
        plsc.store_scatter(cursor, [d], base + cnt, mask=last)
        return carry

    lax.fori_loop(slo // 16, (shi + 15) // 16, dh_body, 0)

    def dpfx(i, running):
        v = cursor[pl.ds(i * 16, 16)]
        vp = (v + 15) & ~15
        c = plsc.cumsum(vp)
        runv = _bcast(running)
        excl = runv + c - vp
        cursor[pl.ds(i * 16, 16)] = excl
        posbuf[pl.ds(i * 16, 16)] = excl
        return running + c[15]

    lax.fori_loop(0, 2, dpfx, jnp.int32(0))

    def dwin(w, ggn):
        s = starts[w]
        e = starts[w + 1]

        def dscat(g, ggn2):
            sl = _bcast(g * 16) + i16b
            inm = (sl >= s) & (sl < e)
            v = sorted_pk[pl.ds(pl.multiple_of(g * 16, 16), 16)]
            d = lax.shift_right_logical(v, 19)
            cnt, last = plsc.scan_count(d, inm)
            base = plsc.load_gather(cursor, [d])
            slot = base + cnt - 1
            pack = (lax.shift_left(lax.shift_right_logical(v, 10), 15)
                    | (_bcast(ggn2 * 16) + i16b))
            plsc.store_scatter(idxbuf, [slot], pack, mask=inm)
            plsc.store_scatter(cursor, [d], base + cnt, mask=last)
            return ggn2 + 1

        return lax.fori_loop(s // 16, (e + 15) // 16, dscat, ggn)

    ggn = lax.fori_loop(wlo, wlo + WPH, dwin, jnp.int32(0))
    ggn = lax.fori_loop(0, lax.select(h == 1, 1, 0),
                        lambda _, gg: dwin(NFULL, gg), ggn)

    posbuf[pl.ds(32, 16)] = cursor[pl.ds(0, 16)]
    posbuf[pl.ds(48, 16)] = cursor[pl.ds(16, 16)]
    pltpu.sync_copy(posbuf.at[pl.ds(0, 64)], dmeta.at[pl.ds(wid * 64, 64)])
    pltpu.sync_copy(idxbuf, dperm_list.at[pl.ds(wid * DCAP, DCAP)])


NSLOT = 16


@functools.partial(
    pl.kernel,
    out_type=(
        jax.ShapeDtypeStruct((BATCH * D_USER,), jnp.float32),
        jax.ShapeDtypeStruct((BATCH * D_ARTIST,), jnp.float32),
    ),
    mesh=_sc_mesh,
    scratch_types=(
        pltpu.VMEM((B_PER_W * 64,), jnp.float32),
        pltpu.VMEM((DCHW,), jnp.int32),
        pltpu.VMEM((NSLOT * 128,), jnp.float32),
        pltpu.VMEM((NSLOT * 16,), jnp.int32),
        pltpu.VMEM((NSLOT * 16,), jnp.int32),
        pltpu.VMEM((NW * 64,), jnp.int32),
        pltpu.SemaphoreType.DMA,
        pltpu.SemaphoreType.DMA,
    ),
    compiler_params=pltpu.CompilerParams(needs_layout_passes=False),
)
def _sc_distribute(val_list, dperm_list, dmeta_hbm,
                   out_u, out_a,
                   rows, seg, ring, stash, stash2, dm, semA, rsem):
    s_id = lax.axis_index("s")
    c_id = lax.axis_index("c")
    wid = s_id * NC + c_id
    base = wid * B_PER_W
    i16 = _I16()

    pltpu.sync_copy(dmeta_hbm, dm)

    def proc_oldest(c):
        nfired, nproc = c
        pltpu.make_async_copy(
            val_list.at[pl.ds(0, 128)], ring.at[pl.ds(0, 128)], rsem).wait()
        slot = lax.bitwise_and(nproc, NSLOT - 1)
        ov = stash[pl.ds(slot * 16, 16)]
        gb = stash2[pl.ds(slot * 16, 16)]
        v8 = plsc.load_gather(ring, [gb + i16 * 16], mask=i16 < 8)
        plsc.store_scatter(rows, [ov], v8, mask=i16 < 8)
        return nfired, nproc + 1

    def drain(c):
        def d_cond(cc):
            nf, np_ = cc
            return np_ < nf

        return lax.while_loop(d_cond, proc_oldest, c)

    def do_table(t, out_flat):
        def src_body(k, carry):
            src_s = t + 2 * lax.shift_right_logical(k, 1)
            src = src_s * NC + lax.bitwise_and(k, 1)
            trf = lax.shift_right_logical(src_s, 1) * 8
            stm = plsc.load_gather(dm, [_bcast(src * 64 + wid)])
            enm = plsc.load_gather(dm, [_bcast(src * 64 + 32 + wid)])
            st = stm[0]
            en = enm[0]
            seglen = en - st
            nchk = (seglen + DCHW - 1) // DCHW
            dbase = src * DCAP + st
            vbase = src * (GCAP * 128)

            def chunk_body(cb, c2):
                pltpu.sync_copy(
                    dperm_list.at[pl.ds(
                        pl.multiple_of(dbase + cb * DCHW, 16), DCHW)], seg)
                lim = lax.min(jnp.int32(DCHW), seglen - cb * DCHW)

                def h_body(g, c3):
                    lv = seg[pl.ds(pl.multiple_of(g * 16, 16), 16)]
                    hi = lax.min(jnp.int32(16), lim - g * 16)
                    for l in range(16):

                        def do_hit(_, c4):
                            c4 = lax.cond(
                                c4[0] - c4[1] == NSLOT, proc_oldest,
                                lambda cc: cc, c4)
                            nfired, nproc = c4
                            loc = lv[l]
                            pos = lax.shift_right_logical(loc, 15)
                            ggn = lax.shift_right_logical(loc & 32767, 4)
                            lane = loc & 15
                            slot = lax.bitwise_and(nfired, NSLOT - 1)
                            offb = (pos - base) * 64 + trf
                            stash[pl.ds(slot * 16, 16)] = jnp.where(
                                i16 < 8, _bcast(offb) + i16, jnp.int32(-1))
                            stash2[pl.ds(slot * 16, 16)] = _bcast(
                                slot * 128 + lane)
                            pltpu.async_copy(
                                val_list.at[pl.ds(
                                    pl.multiple_of(vbase + ggn * 128, 128),
                                    128)],
                                ring.at[pl.ds(slot * 128, 128)], rsem)
                            return nfired + 1, nproc

                        c3 = lax.fori_loop(
                            0, lax.select(l < hi, 1, 0), do_hit, c3)
                    return c3

                return lax.fori_loop(0, (lim + 15) // 16, h_body, c2)

            return lax.fori_loop(0, nchk, chunk_body, carry)

        carry = lax.fori_loop(0, 16, src_body, (jnp.int32(0), jnp.int32(0)))
        carry = drain(carry)
        pltpu.sync_copy(rows, out_flat.at[pl.ds(base * 64, B_PER_W * 64)])
        return carry

    c = do_table(0, out_u)
    del c
    do_table(1, out_a)


def _mlp_body(x_ref, w1_ref, b1_ref, w2_ref, b2_ref, o_ref):
    hh = lax.dot_general(x_ref[:], w1_ref[:], (((1,), (1,)), ((), ())),
                         preferred_element_type=jnp.float32)
    hh = jnp.maximum(hh + b1_ref[:], 0.0)
    o = lax.dot_general(hh, w2_ref[:], (((1,), (1,)), ((), ())),
                        preferred_element_type=jnp.float32)
    o_ref[:] = o + b2_ref[:]


_MLP_BLK = 1024


@jax.jit
def _mlp(audio_features, W1, b1, W2, b2):
    grid = (BATCH // _MLP_BLK,)
    return pl.pallas_call(
        _mlp_body,
        grid=grid,
        in_specs=[
            pl.BlockSpec((_MLP_BLK, 128), lambda i: (i, 0)),
            pl.BlockSpec((256, 128), lambda i: (0, 0)),
            pl.BlockSpec((1, 256), lambda i: (0, 0)),
            pl.BlockSpec((128, 256), lambda i: (0, 0)),
            pl.BlockSpec((1, 128), lambda i: (0, 0)),
        ],
        out_specs=pl.BlockSpec((_MLP_BLK, 128), lambda i: (i, 0)),
        out_shape=jax.ShapeDtypeStruct((BATCH, 128), jnp.float32),
    )(audio_features, W1, b1.reshape(1, 256), W2, b2.reshape(1, 128))


@jax.jit
def kernel(user_ids, genders, genres, artist_ids, audio_features,
           user_table, gender_table, genre_table, artist_table,
           W1, b1, W2, b2):
    ut_t = user_table.T
    at_t = artist_table.T
    ids_ua = jnp.concatenate(
        [user_ids.astype(jnp.int32), artist_ids.astype(jnp.int32)])
    tails = jnp.concatenate(
        [jnp.pad(ut_t[:, LAST0:], ((0, 0), (0, 64))),
         jnp.pad(at_t[:, LAST0:], ((0, 0), (0, 64)))], axis=0)
    gnt_pad = jnp.pad(genre_table.T, ((0, 0), (0, 24)))
    gt_pad = jnp.pad(gender_table.T, ((0, 0), (0, 124)))

    val_list, dperm_list, dmeta, g_t, gn_t = _sc_harvest(
        ids_ua, genders.astype(jnp.int32), genres.astype(jnp.int32),
        ut_t, at_t, gt_pad, gnt_pad, tails)

    u_flat, a_flat = _sc_distribute(val_list, dperm_list, dmeta)

    audio_emb = _mlp(audio_features, W1, b1, W2, b2)
    return (u_flat.reshape(BATCH, D_USER),
            g_t.T,
            gn_t.T,
            a_flat.reshape(BATCH, D_ARTIST),
            audio_emb)
